# baseline (speedup 1.0000x reference)
import functools

import jax
import jax.numpy as jnp
from jax.experimental import pallas as pl
from jax.experimental.pallas import tpu as pltpu

TOPK = 2
NE = 16
D = 2048
BT = 1024
HB = BT // 2
NBUF = 4


def _start_copy(x_hbm, buf, sems, chunk, slot):
    for half in range(2):
        pltpu.make_async_copy(
            x_hbm.at[pl.ds(chunk * BT + half * HB, HB)],
            buf.at[slot, pl.ds(half * HB, HB)],
            sems.at[slot, half],
        ).start()


def _router_body(x_hbm, w_ref, b_ref, scores_ref, wts_ref, idx_ref, buf, sems):
    i = pl.program_id(0)
    nc = pl.num_programs(0)

    @pl.when(i == 0)
    def _prologue():
        for c in range(NBUF):
            _start_copy(x_hbm, buf, sems, c, c)

    slot = jax.lax.rem(i, NBUF)

    def _process_half(half):
        pltpu.make_async_copy(
            x_hbm.at[pl.ds(i * BT + half * HB, HB)],
            buf.at[slot, pl.ds(half * HB, HB)],
            sems.at[slot, half],
        ).wait()
        xb = buf[slot, pl.ds(half * HB, HB)]
        wt = w_ref[...]
        logits = jax.lax.dot_general(
            xb, wt, (((1,), (1,)), ((), ())), preferred_element_type=jnp.float32
        )
        logits = logits + b_ref[...]
        m = jnp.max(logits, axis=1, keepdims=True)
        e = jnp.exp(logits - m)
        p = e / jnp.sum(e, axis=1, keepdims=True)
        scores_ref[pl.ds(half * HB, HB), :] = p

        lane = jax.lax.broadcasted_iota(jnp.int32, p.shape, 1)
        m1 = jnp.max(p, axis=1, keepdims=True)
        i1 = jnp.min(jnp.where(p == m1, lane, NE), axis=1, keepdims=True)
        p2 = jnp.where(lane == i1, -1.0, p)
        m2 = jnp.max(p2, axis=1, keepdims=True)
        i2 = jnp.min(jnp.where(p2 == m2, lane, NE), axis=1, keepdims=True)

        wts_ref[:, pl.ds(half * HB, HB)] = jnp.concatenate([m1, m2], axis=1).T
        idx_ref[:, pl.ds(half * HB, HB)] = jnp.concatenate([i1, i2], axis=1).T

    _process_half(0)
    _process_half(1)

    @pl.when(i + NBUF < nc)
    def _refill():
        _start_copy(x_hbm, buf, sems, i + NBUF, slot)


@functools.partial(jax.jit, static_argnames=("interpret",))
def kernel(x, W, expert_biases, interpret=False):
    batch_shape = x.shape[:-1]
    flat_x = x.reshape(-1, x.shape[-1])
    nt = flat_x.shape[0]
    grid = (nt // BT,)
    bias2d = expert_biases.reshape(1, NE)

    scores, wts, idx = pl.pallas_call(
        _router_body,
        grid=grid,
        in_specs=[
            pl.BlockSpec(memory_space=pl.ANY),
            pl.BlockSpec((NE, D), lambda i: (0, 0)),
            pl.BlockSpec((1, NE), lambda i: (0, 0)),
        ],
        out_specs=[
            pl.BlockSpec((BT, NE), lambda i: (i, 0)),
            pl.BlockSpec((TOPK, BT), lambda i: (0, i)),
            pl.BlockSpec((TOPK, BT), lambda i: (0, i)),
        ],
        out_shape=[
            jax.ShapeDtypeStruct((nt, NE), jnp.float32),
            jax.ShapeDtypeStruct((TOPK, nt), jnp.float32),
            jax.ShapeDtypeStruct((TOPK, nt), jnp.int32),
        ],
        scratch_shapes=[
            pltpu.VMEM((NBUF, BT, D), jnp.float32),
            pltpu.SemaphoreType.DMA((NBUF, 2)),
        ],
        interpret=interpret,
    )(flat_x, W, bias2d)

    return (
        scores.reshape(*batch_shape, NE),
        wts.T.reshape(*batch_shape, TOPK),
        idx.T.reshape(*batch_shape, TOPK),
    )


# speedup vs baseline: 1.3526x; 1.3526x over previous
import functools

import jax
import jax.numpy as jnp
from jax.experimental import pallas as pl
from jax.experimental.pallas import tpu as pltpu

TOPK = 2
NE = 16
D = 2048
BT = 1024
NBUF = 4


def _start_copy(x_hbm, buf, sems, chunk, slot):
    pltpu.make_async_copy(
        x_hbm.at[pl.ds(chunk * BT, BT)], buf.at[slot], sems.at[slot, 0]
    ).start()


def _router_body(x_hbm, w_ref, b_ref, scores_ref, wts_ref, idx_ref, buf, sems):
    i = pl.program_id(0)
    nc = pl.num_programs(0)

    @pl.when(i == 0)
    def _prologue():
        for c in range(NBUF):
            _start_copy(x_hbm, buf, sems, c, c)

    slot = jax.lax.rem(i, NBUF)
    pltpu.make_async_copy(
        x_hbm.at[pl.ds(i * BT, BT)], buf.at[slot], sems.at[slot, 0]
    ).wait()

    xb = buf[slot]
    wt = w_ref[...]
    logits = jax.lax.dot_general(
        xb, wt, (((1,), (1,)), ((), ())), preferred_element_type=jnp.float32
    )
    logits = logits + b_ref[...]
    m = jnp.max(logits, axis=1, keepdims=True)
    e = jnp.exp(logits - m)
    p = e / jnp.sum(e, axis=1, keepdims=True)
    scores_ref[...] = p

    lane = jax.lax.broadcasted_iota(jnp.int32, p.shape, 1)
    m1 = jnp.max(p, axis=1, keepdims=True)
    i1 = jnp.min(jnp.where(p == m1, lane, NE), axis=1, keepdims=True)
    p2 = jnp.where(lane == i1, -1.0, p)
    m2 = jnp.max(p2, axis=1, keepdims=True)
    i2 = jnp.min(jnp.where(p2 == m2, lane, NE), axis=1, keepdims=True)

    wts_ref[...] = jnp.concatenate([m1, m2], axis=1).T
    idx_ref[...] = jnp.concatenate([i1, i2], axis=1).T

    @pl.when(i + NBUF < nc)
    def _refill():
        _start_copy(x_hbm, buf, sems, i + NBUF, slot)


@functools.partial(jax.jit, static_argnames=("interpret",))
def kernel(x, W, expert_biases, interpret=False):
    batch_shape = x.shape[:-1]
    flat_x = x.reshape(-1, x.shape[-1])
    nt = flat_x.shape[0]
    grid = (nt // BT,)
    bias2d = expert_biases.reshape(1, NE)

    scores, wts, idx = pl.pallas_call(
        _router_body,
        grid=grid,
        in_specs=[
            pl.BlockSpec(memory_space=pl.ANY),
            pl.BlockSpec((NE, D), lambda i: (0, 0)),
            pl.BlockSpec((1, NE), lambda i: (0, 0)),
        ],
        out_specs=[
            pl.BlockSpec((BT, NE), lambda i: (i, 0)),
            pl.BlockSpec((TOPK, BT), lambda i: (0, i)),
            pl.BlockSpec((TOPK, BT), lambda i: (0, i)),
        ],
        out_shape=[
            jax.ShapeDtypeStruct((nt, NE), jnp.float32),
            jax.ShapeDtypeStruct((TOPK, nt), jnp.float32),
            jax.ShapeDtypeStruct((TOPK, nt), jnp.int32),
        ],
        scratch_shapes=[
            pltpu.VMEM((NBUF, BT, D), jnp.float32),
            pltpu.SemaphoreType.DMA((NBUF, 2)),
        ],
        interpret=interpret,
    )(flat_x, W, bias2d)

    return (
        scores.reshape(*batch_shape, NE),
        wts.T.reshape(*batch_shape, TOPK),
        idx.T.reshape(*batch_shape, TOPK),
    )


# speedup vs baseline: 1.3703x; 1.0131x over previous
import functools

import jax
import jax.numpy as jnp
from jax.experimental import pallas as pl
from jax.experimental.pallas import tpu as pltpu

TOPK = 2
NE = 16
D = 2048
BT = 1024
NBUF = 4


QS = BT // 4


def _start_copy(x_hbm, buf, sems, chunk, slot):
    for q in range(4):
        pltpu.make_async_copy(
            x_hbm.at[pl.ds(chunk * BT + q * QS, QS)],
            buf.at[slot, pl.ds(q * QS, QS)],
            sems.at[slot, q],
        ).start()


def _router_body(x_hbm, w_ref, b_ref, scores_ref, wts_ref, idx_ref, buf, sems):
    i = pl.program_id(0)
    nc = pl.num_programs(0)

    @pl.when(i == 0)
    def _prologue():
        for c in range(NBUF):
            _start_copy(x_hbm, buf, sems, c, c)

    slot = jax.lax.rem(i, NBUF)
    for q in range(4):
        pltpu.make_async_copy(
            x_hbm.at[pl.ds(i * BT + q * QS, QS)],
            buf.at[slot, pl.ds(q * QS, QS)],
            sems.at[slot, q],
        ).wait()

    xb = buf[slot]
    wt = w_ref[...]
    logits = jax.lax.dot_general(
        xb, wt, (((1,), (1,)), ((), ())), preferred_element_type=jnp.float32
    )
    logits = logits + b_ref[...]
    m = jnp.max(logits, axis=1, keepdims=True)
    e = jnp.exp(logits - m)
    p = e / jnp.sum(e, axis=1, keepdims=True)
    scores_ref[...] = p

    lane = jax.lax.broadcasted_iota(jnp.int32, p.shape, 1)
    m1 = jnp.max(p, axis=1, keepdims=True)
    i1 = jnp.min(jnp.where(p == m1, lane, NE), axis=1, keepdims=True)
    p2 = jnp.where(lane == i1, -1.0, p)
    m2 = jnp.max(p2, axis=1, keepdims=True)
    i2 = jnp.min(jnp.where(p2 == m2, lane, NE), axis=1, keepdims=True)

    wts_ref[...] = jnp.concatenate([m1, m2], axis=1).T
    idx_ref[...] = jnp.concatenate([i1, i2], axis=1).T

    @pl.when(i + NBUF < nc)
    def _refill():
        _start_copy(x_hbm, buf, sems, i + NBUF, slot)


@functools.partial(jax.jit, static_argnames=("interpret",))
def kernel(x, W, expert_biases, interpret=False):
    batch_shape = x.shape[:-1]
    flat_x = x.reshape(-1, x.shape[-1])
    nt = flat_x.shape[0]
    grid = (nt // BT,)
    bias2d = expert_biases.reshape(1, NE)

    scores, wts, idx = pl.pallas_call(
        _router_body,
        grid=grid,
        in_specs=[
            pl.BlockSpec(memory_space=pl.ANY),
            pl.BlockSpec((NE, D), lambda i: (0, 0)),
            pl.BlockSpec((1, NE), lambda i: (0, 0)),
        ],
        out_specs=[
            pl.BlockSpec((BT, NE), lambda i: (i, 0)),
            pl.BlockSpec((TOPK, BT), lambda i: (0, i)),
            pl.BlockSpec((TOPK, BT), lambda i: (0, i)),
        ],
        out_shape=[
            jax.ShapeDtypeStruct((nt, NE), jnp.float32),
            jax.ShapeDtypeStruct((TOPK, nt), jnp.float32),
            jax.ShapeDtypeStruct((TOPK, nt), jnp.int32),
        ],
        scratch_shapes=[
            pltpu.VMEM((NBUF, BT, D), jnp.float32),
            pltpu.SemaphoreType.DMA((NBUF, 4)),
        ],
        interpret=interpret,
    )(flat_x, W, bias2d)

    return (
        scores.reshape(*batch_shape, NE),
        wts.T.reshape(*batch_shape, TOPK),
        idx.T.reshape(*batch_shape, TOPK),
    )
